# baseline (speedup 1.0000x reference)
import functools

import numpy as np
import jax
import jax.numpy as jnp
from jax import lax
from jax.experimental import pallas as pl
from jax.experimental.pallas import tpu as pltpu
from jax.experimental.pallas import tpu_sc as plsc

_N = 16384
_D = 2048
_R = 2811
_NW = 32
_DW = _D // _NW
_C = 256
_NCHUNK = _N // _C
_L = 16


def _build_combine_matrix() -> np.ndarray:
    m = np.zeros((_R, 197), np.float32)
    r = 0
    for mi in range(60):
        for h in range(24):
            m[r, 24 + mi] = 1.0
            m[r, 0 + h] = 1.0
            r += 1
    for s in range(60):
        for mo in range(12):
            m[r, 84 + s] = 1.0
            m[r, 175 + mo] = 1.0
            r += 1
    for d in range(31):
        for y in range(3):
            for w in range(7):
                m[r, 144 + d] = 1.0
                m[r, 187 + y] = 1.0
                m[r, 190 + w] = 1.0
                r += 1
    assert r == _R
    return m


def _build_column_perm() -> np.ndarray:
    orig = np.empty((_D,), np.int64)
    for base in range(0, _D, 32):
        for j in range(16):
            orig[base + 2 * j] = base + j
            orig[base + 2 * j + 1] = base + 16 + j
    return orig


_M_COMBINE = _build_combine_matrix()
_COL_PERM = _build_column_perm()


def _combine_body(m_ref, w_ref, o_ref):
    o_ref[...] = jnp.dot(m_ref[...], w_ref[...],
                         preferred_element_type=jnp.float32
                         ).astype(jnp.bfloat16)


def _combine_tables(m, wcat):
    db = 512
    return pl.pallas_call(
        _combine_body,
        grid=(_D // db,),
        in_specs=[
            pl.BlockSpec((_R, 197), lambda i: (0, 0)),
            pl.BlockSpec((197, db), lambda i: (0, i)),
        ],
        out_specs=pl.BlockSpec((_R, db), lambda i: (0, i)),
        out_shape=jax.ShapeDtypeStruct((_R, _D), jnp.bfloat16),
    )(m, wcat)


def _sc_body(t_hbm, tf_hbm, out_hbm, table_v, tf_v, out_v0, out_v1,
             sem0, sem1):
    cid = lax.axis_index("c")
    sid = lax.axis_index("s")
    wid = sid * 2 + cid
    col0 = wid * _DW

    pltpu.sync_copy(t_hbm.at[:, pl.ds(col0, _DW)], table_v)

    def compute_chunk(k, out_vb, sem_out):
        tok0 = k * _C
        pltpu.sync_copy(tf_hbm.at[:, pl.ds(tok0, _C)], tf_v)

        @plsc.parallel_loop(0, _C, step=_L)
        def _group(base):
            hh = tf_v[0, pl.ds(base, _L)]
            mi = tf_v[1, pl.ds(base, _L)]
            se = tf_v[2, pl.ds(base, _L)]
            dy = tf_v[3, pl.ds(base, _L)]
            mo = tf_v[4, pl.ds(base, _L)]
            yr = tf_v[5, pl.ds(base, _L)]
            wd = tf_v[6, pl.ds(base, _L)]
            i0v = mi * 24 + hh
            i1v = se * 12 + mo + 1439
            i2v = dy * 21 + yr * 7 + wd - 11924
            for l in range(_L):
                a = i0v[l]
                b = i1v[l]
                c = i2v[l]
                for cg in range(_DW // 32):
                    sl = pl.ds(cg * 32, 32)
                    acc = (table_v[a, sl] + table_v[b, sl]) + table_v[c, sl]
                    lo, hi = plsc.unpack(
                        acc, format=plsc.PackFormat.INTERLEAVED)
                    out_vb[base + l, pl.ds(cg * 32, _L)] = lo
                    out_vb[base + l, pl.ds(cg * 32 + _L, _L)] = hi

        pltpu.async_copy(
            out_vb, out_hbm.at[pl.ds(tok0, _C), pl.ds(col0, _DW)], sem_out)

    def chunk_pair(p, _):
        k0 = p * 2

        @pl.when(p >= 1)
        def _():
            pltpu.make_async_copy(
                out_v0, out_hbm.at[pl.ds(0, _C), pl.ds(col0, _DW)],
                sem0).wait()

        compute_chunk(k0, out_v0, sem0)

        @pl.when(p >= 1)
        def _():
            pltpu.make_async_copy(
                out_v1, out_hbm.at[pl.ds(0, _C), pl.ds(col0, _DW)],
                sem1).wait()

        compute_chunk(k0 + 1, out_v1, sem1)
        return 0

    lax.fori_loop(0, _NCHUNK // 2, chunk_pair, 0)
    pltpu.make_async_copy(
        out_v0, out_hbm.at[pl.ds(0, _C), pl.ds(col0, _DW)], sem0).wait()
    pltpu.make_async_copy(
        out_v1, out_hbm.at[pl.ds(0, _C), pl.ds(col0, _DW)], sem1).wait()


def _sc_lookup(table, tf_t):
    mesh = plsc.VectorSubcoreMesh(core_axis_name="c", subcore_axis_name="s")
    run = functools.partial(
        pl.kernel,
        mesh=mesh,
        compiler_params=pltpu.CompilerParams(
            use_tc_tiling_on_sc=False, needs_layout_passes=False),
        out_type=jax.ShapeDtypeStruct((_N, _D), jnp.float32),
        scratch_types=[
            pltpu.VMEM((_R, _DW), jnp.bfloat16),
            pltpu.VMEM((7, _C), jnp.int32),
            pltpu.VMEM((_C, _DW), jnp.float32),
            pltpu.VMEM((_C, _DW), jnp.float32),
            pltpu.SemaphoreType.DMA,
            pltpu.SemaphoreType.DMA,
        ],
    )(_sc_body)
    return run(table, tf_t)


def kernel(time_features, hour_w, minute_w, second_w, day_w, month_w,
           year_w, weekday_w):
    wcat = jnp.concatenate(
        [hour_w, minute_w, second_w, day_w, month_w, year_w, weekday_w],
        axis=0)
    wcat_p = wcat[:, _COL_PERM]
    table = _combine_tables(jnp.asarray(_M_COMBINE), wcat_p)
    tf_t = time_features.T
    return _sc_lookup(table, tf_t)


# speedup vs baseline: 1.7109x; 1.7109x over previous
import functools

import numpy as np
import jax
import jax.numpy as jnp
from jax import lax
from jax.experimental import pallas as pl
from jax.experimental.pallas import tpu as pltpu
from jax.experimental.pallas import tpu_sc as plsc

_N = 16384
_D = 2048
_R = 745
_DW = 128
_TH = _N // 2
_C = 128
_NCHUNK = _TH // _C
_L = 16


def _build_combine_matrix() -> np.ndarray:
    m = np.zeros((_R, 197), np.float32)
    r = 0
    for s in range(60):
        m[r, 84 + s] = 1.0
        r += 1
    for mi in range(60):
        for y in range(3):
            m[r, 24 + mi] = 1.0
            m[r, 187 + y] = 1.0
            r += 1
    for d in range(31):
        for w in range(7):
            m[r, 144 + d] = 1.0
            m[r, 190 + w] = 1.0
            r += 1
    for h in range(24):
        for mo in range(12):
            m[r, 0 + h] = 1.0
            m[r, 175 + mo] = 1.0
            r += 1
    assert r == _R
    return m


_M_COMBINE = _build_combine_matrix()


def _combine_body(m_ref, w_ref, o_ref):
    o_ref[...] = jnp.dot(m_ref[...], w_ref[...],
                         preferred_element_type=jnp.float32)


def _combine_tables(m, wcat):
    db = 512
    return pl.pallas_call(
        _combine_body,
        grid=(_D // db,),
        in_specs=[
            pl.BlockSpec((_R, 197), lambda i: (0, 0)),
            pl.BlockSpec((197, db), lambda i: (0, i)),
        ],
        out_specs=pl.BlockSpec((_R, db), lambda i: (0, i)),
        out_shape=jax.ShapeDtypeStruct((_R, _D), jnp.float32),
    )(m, wcat)


def _sc_body(t_hbm, tf_hbm, out_hbm, table_v, tf_v0, tf_v1, out_v0, out_v1,
             semt0, semt1, semo0, semo1):
    cid = lax.axis_index("c")
    sid = lax.axis_index("s")
    col0 = sid * _DW
    tokb = cid * _TH

    pltpu.sync_copy(t_hbm.at[:, pl.ds(col0, _DW)], table_v)

    def compute_chunk(k, tf_vb, out_vb, sem_out):
        tok0 = tokb + k * _C

        @plsc.parallel_loop(0, _C, step=_L)
        def _group(base):
            hh = tf_vb[0, pl.ds(base, _L)]
            mi = tf_vb[1, pl.ds(base, _L)]
            se = tf_vb[2, pl.ds(base, _L)]
            dy = tf_vb[3, pl.ds(base, _L)]
            mo = tf_vb[4, pl.ds(base, _L)]
            yr = tf_vb[5, pl.ds(base, _L)]
            wd = tf_vb[6, pl.ds(base, _L)]
            i0v = se
            i1v = mi * 3 + yr - 1949
            i2v = dy * 7 + wd + 233
            i3v = hh * 12 + mo + 456
            for l in range(_L):
                a = i0v[l]
                b = i1v[l]
                c = i2v[l]
                d = i3v[l]
                accs = []
                for cg in range(_DW // _L):
                    sl = pl.ds(cg * _L, _L)
                    accs.append((table_v[a, sl] + table_v[b, sl])
                                + (table_v[c, sl] + table_v[d, sl]))
                for cg in range(_DW // _L):
                    out_vb[base + l, pl.ds(cg * _L, _L)] = accs[cg]

        pltpu.async_copy(
            out_vb, out_hbm.at[pl.ds(tok0, _C), pl.ds(col0, _DW)], sem_out)

    def wait_tf(tf_vb, semt):
        pltpu.make_async_copy(
            tf_hbm.at[:, pl.ds(0, _C)], tf_vb, semt).wait()

    def fetch_tf(k, tf_vb, semt):
        pltpu.async_copy(
            tf_hbm.at[:, pl.ds(tokb + k * _C, _C)], tf_vb, semt)

    def wait_out(out_vb, semo):
        pltpu.make_async_copy(
            out_vb, out_hbm.at[pl.ds(0, _C), pl.ds(col0, _DW)], semo).wait()

    fetch_tf(0, tf_v0, semt0)

    def chunk_pair(p, _):
        k0 = p * 2

        wait_tf(tf_v0, semt0)
        fetch_tf(k0 + 1, tf_v1, semt1)

        @pl.when(p >= 1)
        def _():
            wait_out(out_v0, semo0)

        compute_chunk(k0, tf_v0, out_v0, semo0)

        wait_tf(tf_v1, semt1)

        @pl.when(p < _NCHUNK // 2 - 1)
        def _():
            fetch_tf(k0 + 2, tf_v0, semt0)

        @pl.when(p >= 1)
        def _():
            wait_out(out_v1, semo1)

        compute_chunk(k0 + 1, tf_v1, out_v1, semo1)
        return 0

    lax.fori_loop(0, _NCHUNK // 2, chunk_pair, 0)
    wait_out(out_v0, semo0)
    wait_out(out_v1, semo1)


def _sc_lookup(table, tf_t):
    mesh = plsc.VectorSubcoreMesh(core_axis_name="c", subcore_axis_name="s")
    run = functools.partial(
        pl.kernel,
        mesh=mesh,
        out_type=jax.ShapeDtypeStruct((_N, _D), jnp.float32),
        scratch_types=[
            pltpu.VMEM((_R, _DW), jnp.float32),
            pltpu.VMEM((7, _C), jnp.int32),
            pltpu.VMEM((7, _C), jnp.int32),
            pltpu.VMEM((_C, _DW), jnp.float32),
            pltpu.VMEM((_C, _DW), jnp.float32),
            pltpu.SemaphoreType.DMA,
            pltpu.SemaphoreType.DMA,
            pltpu.SemaphoreType.DMA,
            pltpu.SemaphoreType.DMA,
        ],
    )(_sc_body)
    return run(table, tf_t)


def kernel(time_features, hour_w, minute_w, second_w, day_w, month_w,
           year_w, weekday_w):
    wcat = jnp.concatenate(
        [hour_w, minute_w, second_w, day_w, month_w, year_w, weekday_w],
        axis=0)
    table = _combine_tables(jnp.asarray(_M_COMBINE), wcat)
    tf_t = time_features.T
    return _sc_lookup(table, tf_t)
